# Initial kernel scaffold; baseline (speedup 1.0000x reference)
import functools

import jax
import jax.numpy as jnp
from jax import lax
from jax.experimental import pallas as pl
from jax.experimental.pallas import tpu as pltpu, tpu_sc as plsc

_NC, _NS = 2, 16
_NW = _NC * _NS
_CHUNK = 128


def _sc_embed_sum(ex_idx, f_idx, exercise_table, fused_table, n_rows, d):
    rows_per_w = n_rows // _NW
    n_chunks = rows_per_w // _CHUNK
    mesh = plsc.VectorSubcoreMesh(core_axis_name="c", subcore_axis_name="s")

    @functools.partial(
        pl.kernel,
        out_type=jax.ShapeDtypeStruct((n_rows, d), jnp.float32),
        mesh=mesh,
        scratch_types=[
            pltpu.VMEM((n_chunks, _CHUNK), jnp.int32),
            pltpu.VMEM((n_chunks, _CHUNK), jnp.int32),
            pltpu.VMEM((_CHUNK, d), jnp.float32),
            pltpu.SemaphoreType.DMA,
        ],
    )
    def k(ex_idx_hbm, f_idx_hbm, ex_tab_hbm, f_tab_hbm, out_hbm,
          eidx_v, fidx_v, rows_v, sem):
        wid = lax.axis_index("s") * _NC + lax.axis_index("c")
        base = wid * rows_per_w
        pltpu.sync_copy(ex_idx_hbm.at[pl.ds(base, rows_per_w)],
                        eidx_v.reshape(n_chunks * _CHUNK))
        pltpu.sync_copy(f_idx_hbm.at[pl.ds(base, rows_per_w)],
                        fidx_v.reshape(n_chunks * _CHUNK))

        def body(j, carry):
            pltpu.async_copy(f_tab_hbm.at[fidx_v.at[j]], rows_v, sem).wait()
            pltpu.async_copy(ex_tab_hbm.at[eidx_v.at[j]], rows_v, sem,
                             add=True).wait()
            pltpu.sync_copy(rows_v,
                            out_hbm.at[pl.ds(base + j * _CHUNK, _CHUNK)])
            return carry

        lax.fori_loop(0, n_chunks, body, 0)

    return k(ex_idx, f_idx, exercise_table, fused_table)


def kernel(exercises, categories, response, skill, exercise_table,
           position_table, skill_table):
    B, L = exercises.shape
    D = exercise_table.shape[1]
    n_rows = B * L

    fused = (skill_table[:, None, :] + position_table[None, :, :]).reshape(-1, D)

    ex_idx = exercises.reshape(-1).astype(jnp.int32)
    f_idx = (skill.reshape(-1).astype(jnp.int32) * L
             + jnp.tile(jnp.arange(L, dtype=jnp.int32), B))

    out = _sc_embed_sum(ex_idx, f_idx, exercise_table, fused, n_rows, D)
    return out.reshape(B, L, D)


# speedup vs baseline: 2.2361x; 2.2361x over previous
import functools

import jax
import jax.numpy as jnp
from jax import lax
from jax.experimental import pallas as pl
from jax.experimental.pallas import tpu as pltpu, tpu_sc as plsc

_NC, _NS = 2, 16
_NW = _NC * _NS
_CHUNK = 128


def _sc_embed_sum(ex_idx, f_idx, exercise_table, fused_table, n_rows, d):
    rows_per_w = n_rows // _NW
    n_chunks = rows_per_w // _CHUNK
    mesh = plsc.VectorSubcoreMesh(core_axis_name="c", subcore_axis_name="s")

    @functools.partial(
        pl.kernel,
        out_type=jax.ShapeDtypeStruct((n_rows, d), jnp.float32),
        mesh=mesh,
        scratch_types=[
            pltpu.VMEM((n_chunks, _CHUNK), jnp.int32),
            pltpu.VMEM((n_chunks, _CHUNK), jnp.int32),
            pltpu.VMEM((_CHUNK, d), jnp.float32),
            pltpu.SemaphoreType.DMA,
        ],
        compiler_params=pltpu.CompilerParams(use_tc_tiling_on_sc=False),
    )
    def k(ex_idx_hbm, f_idx_hbm, ex_tab_hbm, f_tab_hbm, out_hbm,
          eidx_v, fidx_v, rows_v, sem):
        wid = lax.axis_index("s") * _NC + lax.axis_index("c")
        base = wid * rows_per_w
        pltpu.sync_copy(ex_idx_hbm.at[wid], eidx_v)
        pltpu.sync_copy(f_idx_hbm.at[wid], fidx_v)

        def body(j, carry):
            pltpu.async_copy(f_tab_hbm.at[fidx_v.at[j]], rows_v, sem).wait()
            pltpu.async_copy(ex_tab_hbm.at[eidx_v.at[j]], rows_v, sem,
                             add=True).wait()
            pltpu.sync_copy(rows_v,
                            out_hbm.at[pl.ds(base + j * _CHUNK, _CHUNK)])
            return carry

        lax.fori_loop(0, n_chunks, body, 0)

    return k(ex_idx, f_idx, exercise_table, fused_table)


def kernel(exercises, categories, response, skill, exercise_table,
           position_table, skill_table):
    B, L = exercises.shape
    D = exercise_table.shape[1]
    n_rows = B * L

    fused = (skill_table[:, None, :] + position_table[None, :, :]).reshape(-1, D)

    n_chunks = n_rows // (_NW * _CHUNK)
    ex_idx = exercises.reshape(-1).astype(jnp.int32).reshape(
        _NW, n_chunks, _CHUNK)
    f_idx = (skill.reshape(-1).astype(jnp.int32) * L
             + jnp.tile(jnp.arange(L, dtype=jnp.int32), B)).reshape(
        _NW, n_chunks, _CHUNK)

    out = _sc_embed_sum(ex_idx, f_idx, exercise_table, fused, n_rows, D)
    return out.reshape(B, L, D)


# speedup vs baseline: 2.6805x; 1.1987x over previous
import functools

import jax
import jax.numpy as jnp
from jax import lax
from jax.experimental import pallas as pl
from jax.experimental.pallas import tpu as pltpu, tpu_sc as plsc

_NC, _NS = 2, 16
_NW = _NC * _NS
_CHUNK = 128


_NBUF = 4


def _sc_embed_sum(ex_idx, f_idx, exercise_table, fused_table, n_rows, d):
    rows_per_w = n_rows // _NW
    n_chunks = rows_per_w // _CHUNK
    n_groups = n_chunks // _NBUF
    mesh = plsc.VectorSubcoreMesh(core_axis_name="c", subcore_axis_name="s")

    @functools.partial(
        pl.kernel,
        out_type=jax.ShapeDtypeStruct((n_rows, d), jnp.float32),
        mesh=mesh,
        scratch_types=[
            pltpu.VMEM((n_chunks, _CHUNK), jnp.int32),
            pltpu.VMEM((n_chunks, _CHUNK), jnp.int32),
            [pltpu.VMEM((_CHUNK, d), jnp.float32) for _ in range(_NBUF)],
            pltpu.SemaphoreType.DMA,
            pltpu.SemaphoreType.DMA,
            pltpu.SemaphoreType.DMA,
        ],
        compiler_params=pltpu.CompilerParams(use_tc_tiling_on_sc=False),
    )
    def k(ex_idx_hbm, f_idx_hbm, ex_tab_hbm, f_tab_hbm, out_hbm,
          eidx_v, fidx_v, bufs, sem_f, sem_e, sem_s):
        wid = lax.axis_index("s") * _NC + lax.axis_index("c")
        base = wid * rows_per_w
        pltpu.sync_copy(ex_idx_hbm.at[wid], eidx_v)
        pltpu.sync_copy(f_idx_hbm.at[wid], fidx_v)

        def body(g, carry):
            j0 = g * _NBUF
            fc = [pltpu.async_copy(f_tab_hbm.at[fidx_v.at[j0 + b]], bufs[b],
                                   sem_f) for b in range(_NBUF)]
            ec = []
            for b in range(_NBUF):
                fc[b].wait()
                ec.append(pltpu.async_copy(ex_tab_hbm.at[eidx_v.at[j0 + b]],
                                           bufs[b], sem_e, add=True))
            sc = []
            for b in range(_NBUF):
                ec[b].wait()
                sc.append(pltpu.async_copy(
                    bufs[b],
                    out_hbm.at[pl.ds(base + (j0 + b) * _CHUNK, _CHUNK)],
                    sem_s))
            for b in range(_NBUF):
                sc[b].wait()
            return carry

        lax.fori_loop(0, n_groups, body, 0)

    return k(ex_idx, f_idx, exercise_table, fused_table)


def kernel(exercises, categories, response, skill, exercise_table,
           position_table, skill_table):
    B, L = exercises.shape
    D = exercise_table.shape[1]
    n_rows = B * L

    fused = (skill_table[:, None, :] + position_table[None, :, :]).reshape(-1, D)

    n_chunks = n_rows // (_NW * _CHUNK)
    ex_idx = exercises.reshape(-1).astype(jnp.int32).reshape(
        _NW, n_chunks, _CHUNK)
    f_idx = (skill.reshape(-1).astype(jnp.int32) * L
             + jnp.tile(jnp.arange(L, dtype=jnp.int32), B)).reshape(
        _NW, n_chunks, _CHUNK)

    out = _sc_embed_sum(ex_idx, f_idx, exercise_table, fused, n_rows, D)
    return out.reshape(B, L, D)


# speedup vs baseline: 2.6959x; 1.0057x over previous
import functools

import jax
import jax.numpy as jnp
from jax import lax
from jax.experimental import pallas as pl
from jax.experimental.pallas import tpu as pltpu, tpu_sc as plsc

_NC, _NS = 2, 16
_NW = _NC * _NS
_CHUNK = 128


_NBUF = 8


def _sc_embed_sum(ex_idx, f_idx, exercise_table, fused_table, n_rows, d):
    rows_per_w = n_rows // _NW
    n_chunks = rows_per_w // _CHUNK
    n_groups = n_chunks // _NBUF
    mesh = plsc.VectorSubcoreMesh(core_axis_name="c", subcore_axis_name="s")

    @functools.partial(
        pl.kernel,
        out_type=jax.ShapeDtypeStruct((n_rows, d), jnp.float32),
        mesh=mesh,
        scratch_types=[
            pltpu.VMEM((n_chunks, _CHUNK), jnp.int32),
            pltpu.VMEM((n_chunks, _CHUNK), jnp.int32),
            [pltpu.VMEM((_CHUNK, d), jnp.float32) for _ in range(_NBUF)],
            [pltpu.SemaphoreType.DMA for _ in range(_NBUF)],
            [pltpu.SemaphoreType.DMA for _ in range(_NBUF)],
            [pltpu.SemaphoreType.DMA for _ in range(_NBUF)],
        ],
        compiler_params=pltpu.CompilerParams(use_tc_tiling_on_sc=False),
    )
    def k(ex_idx_hbm, f_idx_hbm, ex_tab_hbm, f_tab_hbm, out_hbm,
          eidx_v, fidx_v, bufs, sems_f, sems_e, sems_s):
        wid = lax.axis_index("s") * _NC + lax.axis_index("c")
        base = wid * rows_per_w
        pltpu.sync_copy(ex_idx_hbm.at[wid], eidx_v)
        pltpu.sync_copy(f_idx_hbm.at[wid], fidx_v)

        def drain_store(b):
            pltpu.make_async_copy(
                bufs[b], out_hbm.at[pl.ds(base, _CHUNK)], sems_s[b]).wait()

        def body(g, carry):
            j0 = g * _NBUF
            for b in range(_NBUF):
                @pl.when(g > 0)
                def _():
                    drain_store(b)
                pltpu.async_copy(f_tab_hbm.at[fidx_v.at[j0 + b]], bufs[b],
                                 sems_f[b])
            for b in range(_NBUF):
                pltpu.make_async_copy(f_tab_hbm.at[fidx_v.at[j0 + b]],
                                      bufs[b], sems_f[b]).wait()
                pltpu.async_copy(ex_tab_hbm.at[eidx_v.at[j0 + b]], bufs[b],
                                 sems_e[b], add=True)
            for b in range(_NBUF):
                pltpu.make_async_copy(ex_tab_hbm.at[eidx_v.at[j0 + b]],
                                      bufs[b], sems_e[b]).wait()
                pltpu.async_copy(
                    bufs[b],
                    out_hbm.at[pl.ds(base + (j0 + b) * _CHUNK, _CHUNK)],
                    sems_s[b])
            return carry

        lax.fori_loop(0, n_groups, body, 0)
        for b in range(_NBUF):
            drain_store(b)

    return k(ex_idx, f_idx, exercise_table, fused_table)


def kernel(exercises, categories, response, skill, exercise_table,
           position_table, skill_table):
    B, L = exercises.shape
    D = exercise_table.shape[1]
    n_rows = B * L

    fused = (skill_table[:, None, :] + position_table[None, :, :]).reshape(-1, D)

    n_chunks = n_rows // (_NW * _CHUNK)
    ex_idx = exercises.reshape(-1).astype(jnp.int32).reshape(
        _NW, n_chunks, _CHUNK)
    f_idx = (skill.reshape(-1).astype(jnp.int32) * L
             + jnp.tile(jnp.arange(L, dtype=jnp.int32), B)).reshape(
        _NW, n_chunks, _CHUNK)

    out = _sc_embed_sum(ex_idx, f_idx, exercise_table, fused, n_rows, D)
    return out.reshape(B, L, D)
